# Initial kernel scaffold; baseline (speedup 1.0000x reference)
import jax
import jax.numpy as jnp
from jax.experimental import pallas as pl


def kernel(x, edge_index, edge_weight, W1, b1, W2, b2):
    raise NotImplementedError("write your pallas kernel here")


# speedup vs baseline: 11.9412x; 11.9412x over previous
import functools

import jax
import jax.numpy as jnp
from jax import lax
from jax.experimental import pallas as pl
from jax.experimental.pallas import tpu as pltpu
from jax.experimental.pallas import tpu_sc as plsc

N = 10000
D = 128
NC = 2
NS = 16
NW = NC * NS
CHUNK = 128
RPT = 624
TAIL = N - RPT * NS


def _sc_mesh():
    return plsc.VectorSubcoreMesh(
        core_axis_name="c", subcore_axis_name="s",
        num_cores=NC, num_subcores=NS)


def _make_deg_kernel(nch):
    @functools.partial(
        pl.kernel,
        out_type=jax.ShapeDtypeStruct((NC, N), jnp.float32),
        mesh=_sc_mesh(),
        scratch_types=[
            pltpu.VMEM((nch, CHUNK), jnp.int32),
            pltpu.VMEM((nch, CHUNK), jnp.float32),
            pltpu.VMEM_SHARED((N,), jnp.float32),
        ],
    )
    def deg_k(col_hbm, ew_hbm, zero_hbm, out_hbm, col_v, ew_v, acc):
        cid = lax.axis_index("c")
        sid = lax.axis_index("s")
        w = sid * NC + cid

        @pl.when(sid == 0)
        def _():
            pltpu.sync_copy(zero_hbm, acc)

        pltpu.sync_copy(col_hbm.at[w], col_v)
        pltpu.sync_copy(ew_hbm.at[w], ew_v)
        plsc.subcore_barrier()

        def body(j, carry):
            pltpu.sync_copy(ew_v.at[j], acc.at[col_v.at[j]], add=True)
            return carry

        lax.fori_loop(0, nch, body, 0)
        plsc.subcore_barrier()

        @pl.when(sid == 0)
        def _():
            pltpu.sync_copy(acc, out_hbm.at[cid])

    return deg_k


def _make_scatter_kernel(nch):
    @functools.partial(
        pl.kernel,
        out_type=jax.ShapeDtypeStruct((NC, N, D), jnp.float32),
        mesh=_sc_mesh(),
        scratch_types=[
            pltpu.VMEM((nch, CHUNK), jnp.int32),
            pltpu.VMEM((nch, CHUNK), jnp.int32),
            pltpu.VMEM((nch, CHUNK), jnp.float32),
            pltpu.VMEM((CHUNK, D), jnp.float32),
            pltpu.VMEM_SHARED((N, D), jnp.float32),
            pltpu.SemaphoreType.DMA,
        ],
    )
    def scat_k(y_hbm, row_hbm, col_hbm, ew_hbm, zero_hbm, out_hbm,
               row_v, col_v, ew_v, buf_a, acc, sem):
        cid = lax.axis_index("c")
        sid = lax.axis_index("s")
        w = sid * NC + cid

        pltpu.sync_copy(zero_hbm.at[pl.ds(sid * RPT, RPT)],
                        acc.at[pl.ds(sid * RPT, RPT)])

        @pl.when(sid == NS - 1)
        def _():
            pltpu.sync_copy(zero_hbm.at[pl.ds(RPT * NS, TAIL)],
                            acc.at[pl.ds(RPT * NS, TAIL)])

        pltpu.sync_copy(row_hbm.at[w], row_v)
        pltpu.sync_copy(col_hbm.at[w], col_v)
        pltpu.sync_copy(ew_hbm.at[w], ew_v)
        plsc.subcore_barrier()

        def chunk_body(j, carry):
            pltpu.async_copy(y_hbm.at[row_v.at[j]], buf_a, sem).wait()

            def grp_body(g, c2):
                ew16 = ew_v[j, pl.ds(g * 16, 16)]
                base = g * 16
                for t in range(16):
                    s = ew16[t]
                    e = base + t
                    for k in range(D // 16):
                        sl = pl.ds(k * 16, 16)
                        buf_a[e, sl] = buf_a[e, sl] * s
                return c2

            lax.fori_loop(0, CHUNK // 16, grp_body, 0)
            pltpu.sync_copy(buf_a, acc.at[col_v.at[j]], add=True)
            return carry

        lax.fori_loop(0, nch, chunk_body, 0)
        plsc.subcore_barrier()
        pltpu.sync_copy(acc.at[pl.ds(sid * RPT, RPT)],
                        out_hbm.at[cid, pl.ds(sid * RPT, RPT)])

        @pl.when(sid == NS - 1)
        def _():
            pltpu.sync_copy(acc.at[pl.ds(RPT * NS, TAIL)],
                            out_hbm.at[cid, pl.ds(RPT * NS, TAIL)])

    return scat_k


def _dinv_of(degt_blk):
    deg = degt_blk[:, 0:1] + degt_blk[:, 1:2] + 1.0
    pos = deg > 0.0
    return jnp.where(pos, lax.rsqrt(jnp.where(pos, deg, 1.0)), 0.0)


def _tc_prescale1(x, w1t, degt):
    def k(x_ref, w_ref, dg_ref, y_ref):
        dinv = _dinv_of(dg_ref[...])
        xw = jnp.dot(x_ref[...], w_ref[...], preferred_element_type=jnp.float32)
        y_ref[...] = xw * dinv

    return pl.pallas_call(
        k, out_shape=jax.ShapeDtypeStruct((N, D), jnp.float32))(x, w1t, degt)


def _tc_mid(sa, sb, y1, degt, b1r, w2t):
    def k(sa_ref, sb_ref, y1_ref, dg_ref, b_ref, w_ref, y2_ref):
        dinv = _dinv_of(dg_ref[...])
        h = dinv * (sa_ref[...] + sb_ref[...] + y1_ref[...]) + b_ref[...]
        h = jnp.maximum(h, 0.0)
        xw2 = jnp.dot(h, w_ref[...], preferred_element_type=jnp.float32)
        y2_ref[...] = xw2 * dinv

    return pl.pallas_call(
        k, out_shape=jax.ShapeDtypeStruct((N, D), jnp.float32))(
            sa, sb, y1, degt, b1r, w2t)


def _tc_final(sa, sb, y2, degt, b2r):
    def k(sa_ref, sb_ref, y2_ref, dg_ref, b_ref, o_ref):
        dinv = _dinv_of(dg_ref[...])
        o_ref[...] = dinv * (sa_ref[...] + sb_ref[...] + y2_ref[...]) + b_ref[...]

    return pl.pallas_call(
        k, out_shape=jax.ShapeDtypeStruct((N, D), jnp.float32))(
            sa, sb, y2, degt, b2r)


def kernel(x, edge_index, edge_weight, W1, b1, W2, b2):
    row = edge_index[0]
    col = edge_index[1]
    e = row.shape[0]
    per = -(-e // NW)
    nch = -(-per // CHUNK)
    pad = nch * CHUNK * NW - e

    row_p = jnp.pad(row, (0, pad)).reshape(NW, nch, CHUNK)
    col_p = jnp.pad(col, (0, pad)).reshape(NW, nch, CHUNK)
    ew_p = jnp.pad(edge_weight, (0, pad)).reshape(NW, nch, CHUNK)
    zero_n = jnp.zeros((N,), jnp.float32)
    zero_nd = jnp.zeros((N, D), jnp.float32)
    w1t = W1.T
    w2t = W2.T
    b1r = b1.reshape(1, D)
    b2r = b2.reshape(1, D)

    deg2 = _make_deg_kernel(nch)(col_p, ew_p, zero_n)
    degt = deg2.T

    scat = _make_scatter_kernel(nch)
    y1 = _tc_prescale1(x, w1t, degt)
    s1 = scat(y1, row_p, col_p, ew_p, zero_nd)
    y2 = _tc_mid(s1[0], s1[1], y1, degt, b1r, w2t)
    s2 = scat(y2, row_p, col_p, ew_p, zero_nd)
    out = _tc_final(s2[0], s2[1], y2, degt, b2r)
    return out
